# baseline (speedup 1.0000x reference)
import functools

import jax
import jax.numpy as jnp
from jax import lax
from jax.experimental import pallas as pl
from jax.experimental.pallas import tpu as pltpu
from jax.experimental.pallas import tpu_sc as plsc

N = 10000
E = 160000
FH = 256
HALF = 128
G = 64
C = 2

CHUNK = 128
NCHUNK = E // CHUNK
NSUB = 16
NCORE = 2
NRC = (N + CHUNK - 1) // CHUNK
TAIL = N - (NRC - 1) * CHUNK


def _sc_degree(dst, ones_c, zeros_c):
    mesh = plsc.VectorSubcoreMesh(core_axis_name="c", subcore_axis_name="s")
    n_per_core = NCHUNK // NCORE

    @functools.partial(
        pl.kernel,
        out_type=jax.ShapeDtypeStruct((NCORE * N,), jnp.float32),
        mesh=mesh,
        scratch_types=[
            pltpu.VMEM((CHUNK,), jnp.int32),
            pltpu.VMEM((CHUNK,), jnp.int32),
            pltpu.VMEM((CHUNK,), jnp.float32),
            pltpu.VMEM((CHUNK,), jnp.float32),
            pltpu.VMEM_SHARED((N,), jnp.float32),
            pltpu.SemaphoreType.DMA,
            pltpu.SemaphoreType.DMA,
        ],
    )
    def k(dst_hbm, ones_hbm, zeros_hbm, out_hbm, didx0, didx1, ones_v,
          stage_v, acc_sh, semd0, semd1):
        c = lax.axis_index("c")
        s = lax.axis_index("s")
        didx = (didx0, didx1)
        semd = (semd0, semd1)
        pltpu.sync_copy(ones_hbm, ones_v)
        pltpu.sync_copy(zeros_hbm, stage_v)

        def init_body(i, carry):
            rc = i * NSUB + s

            @pl.when(rc < NRC - 1)
            def _():
                pltpu.sync_copy(stage_v, acc_sh.at[pl.ds(rc * CHUNK, CHUNK)])

            @pl.when(rc == NRC - 1)
            def _():
                pltpu.sync_copy(stage_v.at[pl.ds(0, TAIL)],
                                acc_sh.at[pl.ds((NRC - 1) * CHUNK, TAIL)])

            return carry

        lax.fori_loop(0, (NRC + NSUB - 1) // NSUB, init_body, 0)
        plsc.subcore_barrier()

        n_iter = (n_per_core + NSUB - 1) // NSUB

        for b in range(2):
            @pl.when(b * NSUB + s < n_per_core)
            def _(b=b):
                e0 = (c * n_per_core + b * NSUB + s) * CHUNK
                pltpu.async_copy(dst_hbm.at[pl.ds(e0, CHUNK)], didx[b],
                                 semd[b])

        for j in range(n_iter):
            b = j % 2
            cid = j * NSUB + s

            @pl.when(cid < n_per_core)
            def _(b=b, j=j, cid=cid):
                pltpu.make_async_copy(dst_hbm.at[pl.ds(0, CHUNK)], didx[b],
                                      semd[b]).wait()
                pltpu.sync_copy(ones_v, acc_sh.at[didx[b]], add=True)
                nxt = (j + 2) * NSUB + s

                @pl.when(nxt < n_per_core)
                def _():
                    e0 = (c * n_per_core + nxt) * CHUNK
                    pltpu.async_copy(dst_hbm.at[pl.ds(e0, CHUNK)], didx[b],
                                     semd[b])

        plsc.subcore_barrier()

        def out_body(i, carry):
            rc = i * NSUB + s

            @pl.when(rc < NRC - 1)
            def _():
                pltpu.sync_copy(acc_sh.at[pl.ds(rc * CHUNK, CHUNK)], stage_v)
                pltpu.sync_copy(stage_v,
                                out_hbm.at[pl.ds(c * N + rc * CHUNK, CHUNK)])

            @pl.when(rc == NRC - 1)
            def _():
                r0 = (NRC - 1) * CHUNK
                pltpu.sync_copy(acc_sh.at[pl.ds(r0, TAIL)],
                                stage_v.at[pl.ds(0, TAIL)])
                pltpu.sync_copy(stage_v.at[pl.ds(0, TAIL)],
                                out_hbm.at[pl.ds(c * N + r0, TAIL)])

            return carry

        lax.fori_loop(0, (NRC + NSUB - 1) // NSUB, out_body, 0)

    return k(dst, ones_c, zeros_c)


PHASE = 40
CPT = NCHUNK // NSUB


def _sc_agg(y_flat, src2, dst, zeros_rows):
    mesh = plsc.VectorSubcoreMesh(core_axis_name="c", subcore_axis_name="s")

    @functools.partial(
        pl.kernel,
        out_type=jax.ShapeDtypeStruct((NCORE * N, HALF), jnp.float32),
        mesh=mesh,
        scratch_types=[
            pltpu.VMEM((PHASE * CHUNK,), jnp.int32),
            pltpu.VMEM((CHUNK,), jnp.int32),
            pltpu.VMEM((CHUNK,), jnp.int32),
            pltpu.VMEM((CHUNK, HALF), jnp.float32),
            pltpu.VMEM((CHUNK, HALF), jnp.float32),
            pltpu.VMEM_SHARED((N, HALF), jnp.float32),
            pltpu.SemaphoreType.DMA,
            pltpu.SemaphoreType.DMA,
            pltpu.SemaphoreType.DMA,
            pltpu.SemaphoreType.DMA,
        ],
    )
    def k(y_hbm, src_hbm, dst_hbm, zeros_hbm, out_hbm,
          sidx_v, didx0, didx1, rows0, rows1, acc_sh,
          semg0, semg1, semd0, semd1):
        c = lax.axis_index("c")
        s = lax.axis_index("s")
        start = s * CPT + jnp.minimum(s, 2)
        count = jnp.where(s < 2, CPT + 1, CPT)
        rows = (rows0, rows1)
        didx = (didx0, didx1)
        semg = (semg0, semg1)
        semd = (semd0, semd1)

        pltpu.sync_copy(zeros_hbm, rows0)
        n_iter_rc = (NRC + NSUB - 1) // NSUB

        def init_body(i, carry):
            rc = i * NSUB + s

            @pl.when(rc < NRC - 1)
            def _():
                pltpu.async_copy(rows0, acc_sh.at[pl.ds(rc * CHUNK, CHUNK)],
                                 semd0)

            @pl.when(rc == NRC - 1)
            def _():
                pltpu.async_copy(rows0.at[pl.ds(0, TAIL)],
                                 acc_sh.at[pl.ds((NRC - 1) * CHUNK, TAIL)],
                                 semd0)

            return carry

        lax.fori_loop(0, n_iter_rc, init_body, 0)

        def init_drain(i, carry):
            rc = i * NSUB + s

            @pl.when(rc < NRC - 1)
            def _():
                pltpu.make_async_copy(rows0, acc_sh.at[pl.ds(0, CHUNK)],
                                      semd0).wait()

            @pl.when(rc == NRC - 1)
            def _():
                pltpu.make_async_copy(rows0.at[pl.ds(0, TAIL)],
                                      acc_sh.at[pl.ds(0, TAIL)],
                                      semd0).wait()

            return carry

        lax.fori_loop(0, n_iter_rc, init_drain, 0)
        plsc.subcore_barrier()

        def do_phase(p):
            base = start + p * PHASE
            cnt = jnp.clip(count - p * PHASE, 0, PHASE)
            pltpu.sync_copy(
                src_hbm.at[pl.ds((c * NCHUNK + base) * CHUNK, PHASE * CHUNK)],
                sidx_v)

            for b in range(2):
                @pl.when(cnt > b)
                def _(b=b):
                    pltpu.async_copy(dst_hbm.at[pl.ds((base + b) * CHUNK,
                                                      CHUNK)],
                                     didx[b], semd[b])
                    pltpu.async_copy(
                        y_hbm.at[sidx_v.at[pl.ds(b * CHUNK, CHUNK)]],
                        rows[b], semg[b])

            def group(g, carry):
                for b in range(2):
                    i = g * 2 + b

                    @pl.when(cnt > i)
                    def _(b=b, i=i):
                        pltpu.make_async_copy(
                            dst_hbm.at[pl.ds(0, CHUNK)], didx[b],
                            semd[b]).wait()
                        pltpu.make_async_copy(
                            y_hbm.at[sidx_v.at[pl.ds(0, CHUNK)]],
                            rows[b], semg[b]).wait()
                        pltpu.sync_copy(rows[b], acc_sh.at[didx[b]],
                                        add=True)

                    @pl.when(cnt > i + 2)
                    def _(b=b, i=i):
                        pltpu.async_copy(
                            dst_hbm.at[pl.ds((base + i + 2) * CHUNK, CHUNK)],
                            didx[b], semd[b])
                        pltpu.async_copy(
                            y_hbm.at[sidx_v.at[pl.ds((i + 2) * CHUNK,
                                                     CHUNK)]],
                            rows[b], semg[b])

                return carry

            lax.fori_loop(0, PHASE // 2, group, 0)

        do_phase(0)
        do_phase(1)
        plsc.subcore_barrier()

        n_full = NRC - 1
        for j in range(n_iter_rc):
            b = j % 2
            rc = j * NSUB + s

            @pl.when(rc < n_full)
            def _(b=b, j=j, rc=rc):
                if j >= 2:
                    pltpu.make_async_copy(rows[b],
                                          out_hbm.at[pl.ds(0, CHUNK)],
                                          semg[b]).wait()
                pltpu.sync_copy(acc_sh.at[pl.ds(rc * CHUNK, CHUNK)], rows[b])
                pltpu.async_copy(rows[b],
                                 out_hbm.at[pl.ds(c * N + rc * CHUNK,
                                                  CHUNK)],
                                 semg[b])

        for j in range(n_iter_rc):
            b = j % 2
            rc = j * NSUB + s
            done = (j + 2) * NSUB + s >= n_full

            @pl.when((rc < n_full) & done)
            def _(b=b):
                pltpu.make_async_copy(rows[b], out_hbm.at[pl.ds(0, CHUNK)],
                                      semg[b]).wait()

        @pl.when(s == (NRC - 1) % NSUB)
        def _():
            r0 = (NRC - 1) * CHUNK
            pltpu.sync_copy(acc_sh.at[pl.ds(r0, TAIL)],
                            rows0.at[pl.ds(0, TAIL)])
            pltpu.sync_copy(rows0.at[pl.ds(0, TAIL)],
                            out_hbm.at[pl.ds(c * N + r0, TAIL)])

    return k(y_flat, src2, dst, zeros_rows)


BR = 5000
NB = N // BR


def _dinv_block(d_ref):
    deg = d_ref[0] + d_ref[1] + 1.0
    return lax.rsqrt(jnp.maximum(deg, 1.0))


def _tc_matmul_scale(xin, W, degp):
    def body(x_ref, w_ref, d_ref, o_ref):
        dinv = _dinv_block(d_ref)
        xw = jnp.dot(x_ref[...], w_ref[...],
                     preferred_element_type=jnp.float32)
        o_ref[0] = xw[:, :HALF] * dinv
        o_ref[1] = xw[:, HALF:] * dinv

    return pl.pallas_call(
        body,
        grid=(NB,),
        in_specs=[
            pl.BlockSpec((BR, FH), lambda r: (r, 0)),
            pl.BlockSpec((FH, FH), lambda r: (0, 0)),
            pl.BlockSpec((2, BR, 1), lambda r: (0, r, 0)),
        ],
        out_specs=pl.BlockSpec((2, BR, HALF), lambda r: (0, r, 0)),
        out_shape=jax.ShapeDtypeStruct((2, N, HALF), jnp.float32),
    )(xin, W, degp)


def _tc_mid(agg3, y3, degp, b, W):
    def body(a_ref, y_ref, d_ref, b_ref, w_ref, o_ref):
        dinv = _dinv_block(d_ref)
        lo = dinv * (a_ref[0] + y_ref[0]) + b_ref[:, :HALF]
        hi = dinv * (a_ref[1] + y_ref[1]) + b_ref[:, HALF:]
        h = jnp.maximum(jnp.concatenate([lo, hi], axis=1), 0.0)
        xw = jnp.dot(h, w_ref[...], preferred_element_type=jnp.float32)
        o_ref[0] = xw[:, :HALF] * dinv
        o_ref[1] = xw[:, HALF:] * dinv

    return pl.pallas_call(
        body,
        grid=(NB,),
        in_specs=[
            pl.BlockSpec((2, BR, HALF), lambda r: (0, r, 0)),
            pl.BlockSpec((2, BR, HALF), lambda r: (0, r, 0)),
            pl.BlockSpec((2, BR, 1), lambda r: (0, r, 0)),
            pl.BlockSpec((1, FH), lambda r: (0, 0)),
            pl.BlockSpec((FH, FH), lambda r: (0, 0)),
        ],
        out_specs=pl.BlockSpec((2, BR, HALF), lambda r: (0, r, 0)),
        out_shape=jax.ShapeDtypeStruct((2, N, HALF), jnp.float32),
    )(agg3, y3, degp, b, W)


def _tc_pool(agg3, y3, degp, b, batch2, Wfc, bfc):
    def body(a_ref, y_ref, d_ref, b_ref, bt_ref, wfc_ref, bfc_ref,
             o_ref, acc, cnt):
        r = pl.program_id(0)

        @pl.when(r == 0)
        def _():
            acc[...] = jnp.zeros((G, FH), jnp.float32)
            cnt[...] = jnp.zeros((G, 1), jnp.float32)

        dinv = _dinv_block(d_ref)
        lo = jnp.maximum(dinv * (a_ref[0] + y_ref[0]) + b_ref[:, :HALF], 0.0)
        hi = jnp.maximum(dinv * (a_ref[1] + y_ref[1]) + b_ref[:, HALF:], 0.0)
        h = jnp.concatenate([lo, hi], axis=1)
        gids = lax.broadcasted_iota(jnp.int32, (BR, G), 1)
        oh = (bt_ref[...] == gids).astype(jnp.float32)
        acc[...] += lax.dot_general(oh, h, (((0,), (0,)), ((), ())),
                                    preferred_element_type=jnp.float32)
        cnt[...] += lax.dot_general(oh, jnp.ones((BR, 1), jnp.float32),
                                    (((0,), (0,)), ((), ())),
                                    preferred_element_type=jnp.float32)

        @pl.when(r == NB - 1)
        def _():
            pooled = acc[...] / jnp.maximum(cnt[...], 1.0)
            logits = jnp.dot(pooled, wfc_ref[...],
                             preferred_element_type=jnp.float32) + bfc_ref[...]
            m = jnp.max(logits, axis=1, keepdims=True)
            z = logits - m
            o_ref[...] = z - jnp.log(jnp.sum(jnp.exp(z), axis=1,
                                             keepdims=True))

    return pl.pallas_call(
        body,
        grid=(NB,),
        in_specs=[
            pl.BlockSpec((2, BR, HALF), lambda r: (0, r, 0)),
            pl.BlockSpec((2, BR, HALF), lambda r: (0, r, 0)),
            pl.BlockSpec((2, BR, 1), lambda r: (0, r, 0)),
            pl.BlockSpec((1, FH), lambda r: (0, 0)),
            pl.BlockSpec((BR, 1), lambda r: (r, 0)),
            pl.BlockSpec((FH, C), lambda r: (0, 0)),
            pl.BlockSpec((1, C), lambda r: (0, 0)),
        ],
        out_specs=pl.BlockSpec((G, C), lambda r: (0, 0)),
        out_shape=jax.ShapeDtypeStruct((G, C), jnp.float32),
        scratch_shapes=[
            pltpu.VMEM((G, FH), jnp.float32),
            pltpu.VMEM((G, 1), jnp.float32),
        ],
    )(agg3, y3, degp, b, batch2, Wfc, bfc)


def kernel(x, edge_index, batch, W1, b1, W2, b2, Wfc, bfc):
    src = edge_index[0]
    dst = edge_index[1]
    src2 = jnp.pad(jnp.concatenate([src, src + N]), (0, 2 * CHUNK))
    ones_c = jnp.ones((CHUNK,), jnp.float32)
    zeros_c = jnp.zeros((CHUNK,), jnp.float32)
    zeros_rows = jnp.zeros((CHUNK, HALF), jnp.float32)

    degp = _sc_degree(dst, ones_c, zeros_c).reshape(2, N, 1)
    y1 = _tc_matmul_scale(x, W1, degp)
    agg1 = _sc_agg(y1.reshape(2 * N, HALF), src2, dst,
                   zeros_rows).reshape(2, N, HALF)
    y2 = _tc_mid(agg1, y1, degp, b1.reshape(1, FH), W2)
    agg2 = _sc_agg(y2.reshape(2 * N, HALF), src2, dst,
                   zeros_rows).reshape(2, N, HALF)

    return _tc_pool(agg2, y2, degp, b2.reshape(1, FH),
                    batch.reshape(N, 1), Wfc, bfc.reshape(1, C))


# speedup vs baseline: 1.0017x; 1.0017x over previous
import functools

import jax
import jax.numpy as jnp
from jax import lax
from jax.experimental import pallas as pl
from jax.experimental.pallas import tpu as pltpu
from jax.experimental.pallas import tpu_sc as plsc

N = 10000
E = 160000
FH = 256
HALF = 128
G = 64
C = 2

CHUNK = 128
NCHUNK = E // CHUNK
NSUB = 16
NCORE = 2
NRC = (N + CHUNK - 1) // CHUNK
TAIL = N - (NRC - 1) * CHUNK


def _sc_degree(dst, ones_c, zeros_c):
    mesh = plsc.VectorSubcoreMesh(core_axis_name="c", subcore_axis_name="s")
    n_per_core = NCHUNK // NCORE

    @functools.partial(
        pl.kernel,
        out_type=jax.ShapeDtypeStruct((NCORE * N,), jnp.float32),
        mesh=mesh,
        scratch_types=[
            pltpu.VMEM((CHUNK,), jnp.int32),
            pltpu.VMEM((CHUNK,), jnp.int32),
            pltpu.VMEM((CHUNK,), jnp.float32),
            pltpu.VMEM((CHUNK,), jnp.float32),
            pltpu.VMEM_SHARED((N,), jnp.float32),
            pltpu.SemaphoreType.DMA,
            pltpu.SemaphoreType.DMA,
        ],
    )
    def k(dst_hbm, ones_hbm, zeros_hbm, out_hbm, didx0, didx1, ones_v,
          stage_v, acc_sh, semd0, semd1):
        c = lax.axis_index("c")
        s = lax.axis_index("s")
        didx = (didx0, didx1)
        semd = (semd0, semd1)
        pltpu.sync_copy(ones_hbm, ones_v)
        pltpu.sync_copy(zeros_hbm, stage_v)

        def init_body(i, carry):
            rc = i * NSUB + s

            @pl.when(rc < NRC - 1)
            def _():
                pltpu.sync_copy(stage_v, acc_sh.at[pl.ds(rc * CHUNK, CHUNK)])

            @pl.when(rc == NRC - 1)
            def _():
                pltpu.sync_copy(stage_v.at[pl.ds(0, TAIL)],
                                acc_sh.at[pl.ds((NRC - 1) * CHUNK, TAIL)])

            return carry

        lax.fori_loop(0, (NRC + NSUB - 1) // NSUB, init_body, 0)
        plsc.subcore_barrier()

        n_iter = (n_per_core + NSUB - 1) // NSUB

        for b in range(2):
            @pl.when(b * NSUB + s < n_per_core)
            def _(b=b):
                e0 = (c * n_per_core + b * NSUB + s) * CHUNK
                pltpu.async_copy(dst_hbm.at[pl.ds(e0, CHUNK)], didx[b],
                                 semd[b])

        for j in range(n_iter):
            b = j % 2
            cid = j * NSUB + s

            @pl.when(cid < n_per_core)
            def _(b=b, j=j, cid=cid):
                pltpu.make_async_copy(dst_hbm.at[pl.ds(0, CHUNK)], didx[b],
                                      semd[b]).wait()
                pltpu.sync_copy(ones_v, acc_sh.at[didx[b]], add=True)
                nxt = (j + 2) * NSUB + s

                @pl.when(nxt < n_per_core)
                def _():
                    e0 = (c * n_per_core + nxt) * CHUNK
                    pltpu.async_copy(dst_hbm.at[pl.ds(e0, CHUNK)], didx[b],
                                     semd[b])

        plsc.subcore_barrier()

        def out_body(i, carry):
            rc = i * NSUB + s

            @pl.when(rc < NRC - 1)
            def _():
                pltpu.sync_copy(acc_sh.at[pl.ds(rc * CHUNK, CHUNK)], stage_v)
                pltpu.sync_copy(stage_v,
                                out_hbm.at[pl.ds(c * N + rc * CHUNK, CHUNK)])

            @pl.when(rc == NRC - 1)
            def _():
                r0 = (NRC - 1) * CHUNK
                pltpu.sync_copy(acc_sh.at[pl.ds(r0, TAIL)],
                                stage_v.at[pl.ds(0, TAIL)])
                pltpu.sync_copy(stage_v.at[pl.ds(0, TAIL)],
                                out_hbm.at[pl.ds(c * N + r0, TAIL)])

            return carry

        lax.fori_loop(0, (NRC + NSUB - 1) // NSUB, out_body, 0)

    return k(dst, ones_c, zeros_c)


PHASE = 40
CPT = NCHUNK // NSUB


def _sc_agg(y_flat, src2, dst, zeros_rows):
    mesh = plsc.VectorSubcoreMesh(core_axis_name="c", subcore_axis_name="s")

    @functools.partial(
        pl.kernel,
        out_type=jax.ShapeDtypeStruct((NCORE * N, HALF), jnp.float32),
        mesh=mesh,
        scratch_types=[
            pltpu.VMEM((PHASE * CHUNK,), jnp.int32),
            pltpu.VMEM((CHUNK,), jnp.int32),
            pltpu.VMEM((CHUNK,), jnp.int32),
            pltpu.VMEM((CHUNK, HALF), jnp.float32),
            pltpu.VMEM((CHUNK, HALF), jnp.float32),
            pltpu.VMEM_SHARED((N, HALF), jnp.float32),
            pltpu.SemaphoreType.DMA,
            pltpu.SemaphoreType.DMA,
            pltpu.SemaphoreType.DMA,
            pltpu.SemaphoreType.DMA,
        ],
    )
    def k(y_hbm, src_hbm, dst_hbm, zeros_hbm, out_hbm,
          sidx_v, didx0, didx1, rows0, rows1, acc_sh,
          semg0, semg1, semd0, semd1):
        c = lax.axis_index("c")
        s = lax.axis_index("s")
        start = s * CPT + jnp.minimum(s, 2)
        count = jnp.where(s < 2, CPT + 1, CPT)
        rows = (rows0, rows1)
        didx = (didx0, didx1)
        semg = (semg0, semg1)
        semd = (semd0, semd1)

        pltpu.sync_copy(zeros_hbm, rows0)
        n_iter_rc = (NRC + NSUB - 1) // NSUB

        def init_body(i, carry):
            rc = i * NSUB + s

            @pl.when(rc < NRC - 1)
            def _():
                pltpu.async_copy(rows0, acc_sh.at[pl.ds(rc * CHUNK, CHUNK)],
                                 semd0)

            @pl.when(rc == NRC - 1)
            def _():
                pltpu.async_copy(rows0.at[pl.ds(0, TAIL)],
                                 acc_sh.at[pl.ds((NRC - 1) * CHUNK, TAIL)],
                                 semd0)

            return carry

        lax.fori_loop(0, n_iter_rc, init_body, 0)

        def init_drain(i, carry):
            rc = i * NSUB + s

            @pl.when(rc < NRC - 1)
            def _():
                pltpu.make_async_copy(rows0, acc_sh.at[pl.ds(0, CHUNK)],
                                      semd0).wait()

            @pl.when(rc == NRC - 1)
            def _():
                pltpu.make_async_copy(rows0.at[pl.ds(0, TAIL)],
                                      acc_sh.at[pl.ds(0, TAIL)],
                                      semd0).wait()

            return carry

        lax.fori_loop(0, n_iter_rc, init_drain, 0)
        plsc.subcore_barrier()

        def do_phase(p):
            base = start + p * PHASE
            cnt = jnp.clip(count - p * PHASE, 0, PHASE)
            pltpu.sync_copy(
                src_hbm.at[pl.ds((c * NCHUNK + base) * CHUNK, PHASE * CHUNK)],
                sidx_v)

            for b in range(2):
                @pl.when(cnt > b)
                def _(b=b):
                    pltpu.async_copy(dst_hbm.at[pl.ds((base + b) * CHUNK,
                                                      CHUNK)],
                                     didx[b], semd[b])
                    pltpu.async_copy(
                        y_hbm.at[sidx_v.at[pl.ds(b * CHUNK, CHUNK)]],
                        rows[b], semg[b])

            def group(g, carry):
                for b in range(2):
                    i = g * 2 + b

                    @pl.when(cnt > i)
                    def _(b=b, i=i):
                        pltpu.make_async_copy(
                            dst_hbm.at[pl.ds(0, CHUNK)], didx[b],
                            semd[b]).wait()
                        pltpu.make_async_copy(
                            y_hbm.at[sidx_v.at[pl.ds(0, CHUNK)]],
                            rows[b], semg[b]).wait()
                        pltpu.sync_copy(rows[b], acc_sh.at[didx[b]],
                                        add=True)

                    @pl.when(cnt > i + 2)
                    def _(b=b, i=i):
                        pltpu.async_copy(
                            dst_hbm.at[pl.ds((base + i + 2) * CHUNK, CHUNK)],
                            didx[b], semd[b])
                        pltpu.async_copy(
                            y_hbm.at[sidx_v.at[pl.ds((i + 2) * CHUNK,
                                                     CHUNK)]],
                            rows[b], semg[b])

                return carry

            lax.fori_loop(0, PHASE // 2, group, 0)

        do_phase(0)
        do_phase(1)
        plsc.subcore_barrier()

        n_full = NRC - 1
        for j in range(n_iter_rc):
            b = j % 2
            rc = j * NSUB + s

            @pl.when(rc < n_full)
            def _(b=b, j=j, rc=rc):
                if j >= 2:
                    pltpu.make_async_copy(rows[b],
                                          out_hbm.at[pl.ds(0, CHUNK)],
                                          semg[b]).wait()
                pltpu.sync_copy(acc_sh.at[pl.ds(rc * CHUNK, CHUNK)], rows[b])
                pltpu.async_copy(rows[b],
                                 out_hbm.at[pl.ds(c * N + rc * CHUNK,
                                                  CHUNK)],
                                 semg[b])

        for j in range(n_iter_rc):
            b = j % 2
            rc = j * NSUB + s
            done = (j + 2) * NSUB + s >= n_full

            @pl.when((rc < n_full) & done)
            def _(b=b):
                pltpu.make_async_copy(rows[b], out_hbm.at[pl.ds(0, CHUNK)],
                                      semg[b]).wait()

        @pl.when(s == (NRC - 1) % NSUB)
        def _():
            r0 = (NRC - 1) * CHUNK
            pltpu.sync_copy(acc_sh.at[pl.ds(r0, TAIL)],
                            rows0.at[pl.ds(0, TAIL)])
            pltpu.sync_copy(rows0.at[pl.ds(0, TAIL)],
                            out_hbm.at[pl.ds(c * N + r0, TAIL)])

    return k(y_flat, src2, dst, zeros_rows)


BR = 2000
NB = N // BR


def _dinv_block(d_ref):
    deg = d_ref[0] + d_ref[1] + 1.0
    return lax.rsqrt(jnp.maximum(deg, 1.0))


def _tc_matmul_scale(xin, W, degp):
    def body(x_ref, w_ref, d_ref, o_ref):
        dinv = _dinv_block(d_ref)
        xw = jnp.dot(x_ref[...], w_ref[...],
                     preferred_element_type=jnp.float32)
        o_ref[0] = xw[:, :HALF] * dinv
        o_ref[1] = xw[:, HALF:] * dinv

    return pl.pallas_call(
        body,
        grid=(NB,),
        in_specs=[
            pl.BlockSpec((BR, FH), lambda r: (r, 0)),
            pl.BlockSpec((FH, FH), lambda r: (0, 0)),
            pl.BlockSpec((2, BR, 1), lambda r: (0, r, 0)),
        ],
        out_specs=pl.BlockSpec((2, BR, HALF), lambda r: (0, r, 0)),
        out_shape=jax.ShapeDtypeStruct((2, N, HALF), jnp.float32),
    )(xin, W, degp)


def _tc_mid(agg3, y3, degp, b, W):
    def body(a_ref, y_ref, d_ref, b_ref, w_ref, o_ref):
        dinv = _dinv_block(d_ref)
        lo = dinv * (a_ref[0] + y_ref[0]) + b_ref[:, :HALF]
        hi = dinv * (a_ref[1] + y_ref[1]) + b_ref[:, HALF:]
        h = jnp.maximum(jnp.concatenate([lo, hi], axis=1), 0.0)
        xw = jnp.dot(h, w_ref[...], preferred_element_type=jnp.float32)
        o_ref[0] = xw[:, :HALF] * dinv
        o_ref[1] = xw[:, HALF:] * dinv

    return pl.pallas_call(
        body,
        grid=(NB,),
        in_specs=[
            pl.BlockSpec((2, BR, HALF), lambda r: (0, r, 0)),
            pl.BlockSpec((2, BR, HALF), lambda r: (0, r, 0)),
            pl.BlockSpec((2, BR, 1), lambda r: (0, r, 0)),
            pl.BlockSpec((1, FH), lambda r: (0, 0)),
            pl.BlockSpec((FH, FH), lambda r: (0, 0)),
        ],
        out_specs=pl.BlockSpec((2, BR, HALF), lambda r: (0, r, 0)),
        out_shape=jax.ShapeDtypeStruct((2, N, HALF), jnp.float32),
    )(agg3, y3, degp, b, W)


def _tc_pool(agg3, y3, degp, b, batch2, Wfc, bfc):
    def body(a_ref, y_ref, d_ref, b_ref, bt_ref, wfc_ref, bfc_ref,
             o_ref, acc, cnt):
        r = pl.program_id(0)

        @pl.when(r == 0)
        def _():
            acc[...] = jnp.zeros((G, FH), jnp.float32)
            cnt[...] = jnp.zeros((G, 1), jnp.float32)

        dinv = _dinv_block(d_ref)
        lo = jnp.maximum(dinv * (a_ref[0] + y_ref[0]) + b_ref[:, :HALF], 0.0)
        hi = jnp.maximum(dinv * (a_ref[1] + y_ref[1]) + b_ref[:, HALF:], 0.0)
        h = jnp.concatenate([lo, hi], axis=1)
        gids = lax.broadcasted_iota(jnp.int32, (BR, G), 1)
        oh = (bt_ref[...] == gids).astype(jnp.float32)
        acc[...] += lax.dot_general(oh, h, (((0,), (0,)), ((), ())),
                                    preferred_element_type=jnp.float32)
        cnt[...] += lax.dot_general(oh, jnp.ones((BR, 1), jnp.float32),
                                    (((0,), (0,)), ((), ())),
                                    preferred_element_type=jnp.float32)

        @pl.when(r == NB - 1)
        def _():
            pooled = acc[...] / jnp.maximum(cnt[...], 1.0)
            logits = jnp.dot(pooled, wfc_ref[...],
                             preferred_element_type=jnp.float32) + bfc_ref[...]
            m = jnp.max(logits, axis=1, keepdims=True)
            z = logits - m
            o_ref[...] = z - jnp.log(jnp.sum(jnp.exp(z), axis=1,
                                             keepdims=True))

    return pl.pallas_call(
        body,
        grid=(NB,),
        in_specs=[
            pl.BlockSpec((2, BR, HALF), lambda r: (0, r, 0)),
            pl.BlockSpec((2, BR, HALF), lambda r: (0, r, 0)),
            pl.BlockSpec((2, BR, 1), lambda r: (0, r, 0)),
            pl.BlockSpec((1, FH), lambda r: (0, 0)),
            pl.BlockSpec((BR, 1), lambda r: (r, 0)),
            pl.BlockSpec((FH, C), lambda r: (0, 0)),
            pl.BlockSpec((1, C), lambda r: (0, 0)),
        ],
        out_specs=pl.BlockSpec((G, C), lambda r: (0, 0)),
        out_shape=jax.ShapeDtypeStruct((G, C), jnp.float32),
        scratch_shapes=[
            pltpu.VMEM((G, FH), jnp.float32),
            pltpu.VMEM((G, 1), jnp.float32),
        ],
    )(agg3, y3, degp, b, batch2, Wfc, bfc)


def kernel(x, edge_index, batch, W1, b1, W2, b2, Wfc, bfc):
    src = edge_index[0]
    dst = edge_index[1]
    src2 = jnp.pad(jnp.concatenate([src, src + N]), (0, 2 * CHUNK))
    ones_c = jnp.ones((CHUNK,), jnp.float32)
    zeros_c = jnp.zeros((CHUNK,), jnp.float32)
    zeros_rows = jnp.zeros((CHUNK, HALF), jnp.float32)

    degp = _sc_degree(dst, ones_c, zeros_c).reshape(2, N, 1)
    y1 = _tc_matmul_scale(x, W1, degp)
    agg1 = _sc_agg(y1.reshape(2 * N, HALF), src2, dst,
                   zeros_rows).reshape(2, N, HALF)
    y2 = _tc_mid(agg1, y1, degp, b1.reshape(1, FH), W2)
    agg2 = _sc_agg(y2.reshape(2 * N, HALF), src2, dst,
                   zeros_rows).reshape(2, N, HALF)

    return _tc_pool(agg2, y2, degp, b2.reshape(1, FH),
                    batch.reshape(N, 1), Wfc, bfc.reshape(1, C))


# speedup vs baseline: 1.0108x; 1.0091x over previous
import functools

import jax
import jax.numpy as jnp
from jax import lax
from jax.experimental import pallas as pl
from jax.experimental.pallas import tpu as pltpu
from jax.experimental.pallas import tpu_sc as plsc

N = 10000
E = 160000
FH = 256
HALF = 128
G = 64
C = 2

CHUNK = 128
NCHUNK = E // CHUNK
NSUB = 16
NCORE = 2
NRC = (N + CHUNK - 1) // CHUNK
TAIL = N - (NRC - 1) * CHUNK


def _sc_degree(dst, ones_c, zeros_c):
    mesh = plsc.VectorSubcoreMesh(core_axis_name="c", subcore_axis_name="s")
    n_per_core = NCHUNK // NCORE

    @functools.partial(
        pl.kernel,
        out_type=jax.ShapeDtypeStruct((NCORE * N,), jnp.float32),
        mesh=mesh,
        scratch_types=[
            pltpu.VMEM((CHUNK,), jnp.int32),
            pltpu.VMEM((CHUNK,), jnp.int32),
            pltpu.VMEM((CHUNK,), jnp.float32),
            pltpu.VMEM((CHUNK,), jnp.float32),
            pltpu.VMEM_SHARED((N,), jnp.float32),
            pltpu.SemaphoreType.DMA,
            pltpu.SemaphoreType.DMA,
        ],
    )
    def k(dst_hbm, ones_hbm, zeros_hbm, out_hbm, didx0, didx1, ones_v,
          stage_v, acc_sh, semd0, semd1):
        c = lax.axis_index("c")
        s = lax.axis_index("s")
        didx = (didx0, didx1)
        semd = (semd0, semd1)
        pltpu.sync_copy(ones_hbm, ones_v)
        pltpu.sync_copy(zeros_hbm, stage_v)

        def init_body(i, carry):
            rc = i * NSUB + s

            @pl.when(rc < NRC - 1)
            def _():
                pltpu.sync_copy(stage_v, acc_sh.at[pl.ds(rc * CHUNK, CHUNK)])

            @pl.when(rc == NRC - 1)
            def _():
                pltpu.sync_copy(stage_v.at[pl.ds(0, TAIL)],
                                acc_sh.at[pl.ds((NRC - 1) * CHUNK, TAIL)])

            return carry

        lax.fori_loop(0, (NRC + NSUB - 1) // NSUB, init_body, 0)
        plsc.subcore_barrier()

        n_iter = (n_per_core + NSUB - 1) // NSUB

        for b in range(2):
            @pl.when(b * NSUB + s < n_per_core)
            def _(b=b):
                e0 = (c * n_per_core + b * NSUB + s) * CHUNK
                pltpu.async_copy(dst_hbm.at[pl.ds(e0, CHUNK)], didx[b],
                                 semd[b])

        for j in range(n_iter):
            b = j % 2
            cid = j * NSUB + s

            @pl.when(cid < n_per_core)
            def _(b=b, j=j, cid=cid):
                pltpu.make_async_copy(dst_hbm.at[pl.ds(0, CHUNK)], didx[b],
                                      semd[b]).wait()
                pltpu.sync_copy(ones_v, acc_sh.at[didx[b]], add=True)
                nxt = (j + 2) * NSUB + s

                @pl.when(nxt < n_per_core)
                def _():
                    e0 = (c * n_per_core + nxt) * CHUNK
                    pltpu.async_copy(dst_hbm.at[pl.ds(e0, CHUNK)], didx[b],
                                     semd[b])

        plsc.subcore_barrier()

        def out_body(i, carry):
            rc = i * NSUB + s

            @pl.when(rc < NRC - 1)
            def _():
                pltpu.sync_copy(acc_sh.at[pl.ds(rc * CHUNK, CHUNK)], stage_v)
                pltpu.sync_copy(stage_v,
                                out_hbm.at[pl.ds(c * N + rc * CHUNK, CHUNK)])

            @pl.when(rc == NRC - 1)
            def _():
                r0 = (NRC - 1) * CHUNK
                pltpu.sync_copy(acc_sh.at[pl.ds(r0, TAIL)],
                                stage_v.at[pl.ds(0, TAIL)])
                pltpu.sync_copy(stage_v.at[pl.ds(0, TAIL)],
                                out_hbm.at[pl.ds(c * N + r0, TAIL)])

            return carry

        lax.fori_loop(0, (NRC + NSUB - 1) // NSUB, out_body, 0)

    return k(dst, ones_c, zeros_c)


PHASE = 40
CPT = NCHUNK // NSUB


def _sc_agg(y_flat, src2, dst):
    mesh = plsc.VectorSubcoreMesh(core_axis_name="c", subcore_axis_name="s")

    @functools.partial(
        pl.kernel,
        out_type=jax.ShapeDtypeStruct((NCORE * N, HALF), jnp.float32),
        mesh=mesh,
        scratch_types=[
            pltpu.VMEM((PHASE * CHUNK,), jnp.int32),
            pltpu.VMEM((CHUNK,), jnp.int32),
            pltpu.VMEM((CHUNK,), jnp.int32),
            pltpu.VMEM((CHUNK, HALF), jnp.float32),
            pltpu.VMEM((CHUNK, HALF), jnp.float32),
            pltpu.VMEM_SHARED((N, HALF), jnp.float32),
            pltpu.SemaphoreType.DMA,
            pltpu.SemaphoreType.DMA,
            pltpu.SemaphoreType.DMA,
            pltpu.SemaphoreType.DMA,
        ],
    )
    def k(y_hbm, src_hbm, dst_hbm, out_hbm,
          sidx_v, didx0, didx1, rows0, rows1, acc_sh,
          semg0, semg1, semd0, semd1):
        c = lax.axis_index("c")
        s = lax.axis_index("s")
        start = s * CPT + jnp.minimum(s, 2)
        count = jnp.where(s < 2, CPT + 1, CPT)
        rows = (rows0, rows1)
        didx = (didx0, didx1)
        semg = (semg0, semg1)
        semd = (semd0, semd1)

        n_iter_rc = (NRC + NSUB - 1) // NSUB

        for j in range(n_iter_rc):
            b = j % 2
            rc = j * NSUB + s

            @pl.when(rc < NRC - 1)
            def _(b=b, j=j, rc=rc):
                if j >= 2:
                    pltpu.make_async_copy(rows[b],
                                          acc_sh.at[pl.ds(0, CHUNK)],
                                          semd[b]).wait()
                pltpu.sync_copy(y_hbm.at[pl.ds(c * N + rc * CHUNK, CHUNK)],
                                rows[b])
                pltpu.async_copy(rows[b], acc_sh.at[pl.ds(rc * CHUNK,
                                                          CHUNK)],
                                 semd[b])

        for j in range(n_iter_rc):
            b = j % 2
            rc = j * NSUB + s
            done = (j + 2) * NSUB + s >= NRC - 1

            @pl.when((rc < NRC - 1) & done)
            def _(b=b):
                pltpu.make_async_copy(rows[b], acc_sh.at[pl.ds(0, CHUNK)],
                                      semd[b]).wait()

        @pl.when(s == (NRC - 1) % NSUB)
        def _():
            r0 = (NRC - 1) * CHUNK
            pltpu.sync_copy(y_hbm.at[pl.ds(c * N + r0, TAIL)],
                            rows0.at[pl.ds(0, TAIL)])
            pltpu.sync_copy(rows0.at[pl.ds(0, TAIL)],
                            acc_sh.at[pl.ds(r0, TAIL)])

        plsc.subcore_barrier()

        def do_phase(p):
            base = start + p * PHASE
            cnt = jnp.clip(count - p * PHASE, 0, PHASE)
            pltpu.sync_copy(
                src_hbm.at[pl.ds((c * NCHUNK + base) * CHUNK, PHASE * CHUNK)],
                sidx_v)

            for b in range(2):
                @pl.when(cnt > b)
                def _(b=b):
                    pltpu.async_copy(dst_hbm.at[pl.ds((base + b) * CHUNK,
                                                      CHUNK)],
                                     didx[b], semd[b])
                    pltpu.async_copy(
                        y_hbm.at[sidx_v.at[pl.ds(b * CHUNK, CHUNK)]],
                        rows[b], semg[b])

            def group(g, carry):
                for b in range(2):
                    i = g * 2 + b

                    @pl.when(cnt > i)
                    def _(b=b, i=i):
                        pltpu.make_async_copy(
                            dst_hbm.at[pl.ds(0, CHUNK)], didx[b],
                            semd[b]).wait()
                        pltpu.make_async_copy(
                            y_hbm.at[sidx_v.at[pl.ds(0, CHUNK)]],
                            rows[b], semg[b]).wait()
                        pltpu.sync_copy(rows[b], acc_sh.at[didx[b]],
                                        add=True)

                    @pl.when(cnt > i + 2)
                    def _(b=b, i=i):
                        pltpu.async_copy(
                            dst_hbm.at[pl.ds((base + i + 2) * CHUNK, CHUNK)],
                            didx[b], semd[b])
                        pltpu.async_copy(
                            y_hbm.at[sidx_v.at[pl.ds((i + 2) * CHUNK,
                                                     CHUNK)]],
                            rows[b], semg[b])

                return carry

            lax.fori_loop(0, PHASE // 2, group, 0)

        do_phase(0)
        do_phase(1)
        plsc.subcore_barrier()

        n_full = NRC - 1
        for j in range(n_iter_rc):
            b = j % 2
            rc = j * NSUB + s

            @pl.when(rc < n_full)
            def _(b=b, j=j, rc=rc):
                if j >= 2:
                    pltpu.make_async_copy(rows[b],
                                          out_hbm.at[pl.ds(0, CHUNK)],
                                          semg[b]).wait()
                pltpu.sync_copy(acc_sh.at[pl.ds(rc * CHUNK, CHUNK)], rows[b])
                pltpu.async_copy(rows[b],
                                 out_hbm.at[pl.ds(c * N + rc * CHUNK,
                                                  CHUNK)],
                                 semg[b])

        for j in range(n_iter_rc):
            b = j % 2
            rc = j * NSUB + s
            done = (j + 2) * NSUB + s >= n_full

            @pl.when((rc < n_full) & done)
            def _(b=b):
                pltpu.make_async_copy(rows[b], out_hbm.at[pl.ds(0, CHUNK)],
                                      semg[b]).wait()

        @pl.when(s == (NRC - 1) % NSUB)
        def _():
            r0 = (NRC - 1) * CHUNK
            pltpu.sync_copy(acc_sh.at[pl.ds(r0, TAIL)],
                            rows0.at[pl.ds(0, TAIL)])
            pltpu.sync_copy(rows0.at[pl.ds(0, TAIL)],
                            out_hbm.at[pl.ds(c * N + r0, TAIL)])

    return k(y_flat, src2, dst)


BR = 2000
NB = N // BR


def _dinv_block(d_ref):
    deg = d_ref[0] + d_ref[1] + 1.0
    return lax.rsqrt(jnp.maximum(deg, 1.0))


def _tc_matmul_scale(xin, W, degp):
    def body(x_ref, w_ref, d_ref, o_ref):
        dinv = _dinv_block(d_ref)
        xw = jnp.dot(x_ref[...], w_ref[...],
                     preferred_element_type=jnp.float32)
        o_ref[0] = xw[:, :HALF] * dinv
        o_ref[1] = xw[:, HALF:] * dinv

    return pl.pallas_call(
        body,
        grid=(NB,),
        in_specs=[
            pl.BlockSpec((BR, FH), lambda r: (r, 0)),
            pl.BlockSpec((FH, FH), lambda r: (0, 0)),
            pl.BlockSpec((2, BR, 1), lambda r: (0, r, 0)),
        ],
        out_specs=pl.BlockSpec((2, BR, HALF), lambda r: (0, r, 0)),
        out_shape=jax.ShapeDtypeStruct((2, N, HALF), jnp.float32),
    )(xin, W, degp)


def _tc_mid(aggy, degp, b, W):
    def body(a_ref, d_ref, b_ref, w_ref, o_ref):
        dinv = _dinv_block(d_ref)
        lo = dinv * a_ref[0] + b_ref[:, :HALF]
        hi = dinv * a_ref[1] + b_ref[:, HALF:]
        h = jnp.maximum(jnp.concatenate([lo, hi], axis=1), 0.0)
        xw = jnp.dot(h, w_ref[...], preferred_element_type=jnp.float32)
        o_ref[0] = xw[:, :HALF] * dinv
        o_ref[1] = xw[:, HALF:] * dinv

    return pl.pallas_call(
        body,
        grid=(NB,),
        in_specs=[
            pl.BlockSpec((2, BR, HALF), lambda r: (0, r, 0)),
            pl.BlockSpec((2, BR, 1), lambda r: (0, r, 0)),
            pl.BlockSpec((1, FH), lambda r: (0, 0)),
            pl.BlockSpec((FH, FH), lambda r: (0, 0)),
        ],
        out_specs=pl.BlockSpec((2, BR, HALF), lambda r: (0, r, 0)),
        out_shape=jax.ShapeDtypeStruct((2, N, HALF), jnp.float32),
    )(aggy, degp, b, W)


def _tc_pool(aggy, degp, b, batch2, Wfc, bfc):
    def body(a_ref, d_ref, b_ref, bt_ref, wfc_ref, bfc_ref,
             o_ref, acc, cnt):
        r = pl.program_id(0)

        @pl.when(r == 0)
        def _():
            acc[...] = jnp.zeros((G, FH), jnp.float32)
            cnt[...] = jnp.zeros((G, 1), jnp.float32)

        dinv = _dinv_block(d_ref)
        lo = jnp.maximum(dinv * a_ref[0] + b_ref[:, :HALF], 0.0)
        hi = jnp.maximum(dinv * a_ref[1] + b_ref[:, HALF:], 0.0)
        h = jnp.concatenate([lo, hi], axis=1)
        gids = lax.broadcasted_iota(jnp.int32, (BR, G), 1)
        oh = (bt_ref[...] == gids).astype(jnp.float32)
        acc[...] += lax.dot_general(oh, h, (((0,), (0,)), ((), ())),
                                    preferred_element_type=jnp.float32)
        cnt[...] += lax.dot_general(oh, jnp.ones((BR, 1), jnp.float32),
                                    (((0,), (0,)), ((), ())),
                                    preferred_element_type=jnp.float32)

        @pl.when(r == NB - 1)
        def _():
            pooled = acc[...] / jnp.maximum(cnt[...], 1.0)
            logits = jnp.dot(pooled, wfc_ref[...],
                             preferred_element_type=jnp.float32) + bfc_ref[...]
            m = jnp.max(logits, axis=1, keepdims=True)
            z = logits - m
            o_ref[...] = z - jnp.log(jnp.sum(jnp.exp(z), axis=1,
                                             keepdims=True))

    return pl.pallas_call(
        body,
        grid=(NB,),
        in_specs=[
            pl.BlockSpec((2, BR, HALF), lambda r: (0, r, 0)),
            pl.BlockSpec((2, BR, 1), lambda r: (0, r, 0)),
            pl.BlockSpec((1, FH), lambda r: (0, 0)),
            pl.BlockSpec((BR, 1), lambda r: (r, 0)),
            pl.BlockSpec((FH, C), lambda r: (0, 0)),
            pl.BlockSpec((1, C), lambda r: (0, 0)),
        ],
        out_specs=pl.BlockSpec((G, C), lambda r: (0, 0)),
        out_shape=jax.ShapeDtypeStruct((G, C), jnp.float32),
        scratch_shapes=[
            pltpu.VMEM((G, FH), jnp.float32),
            pltpu.VMEM((G, 1), jnp.float32),
        ],
    )(aggy, degp, b, batch2, Wfc, bfc)


def kernel(x, edge_index, batch, W1, b1, W2, b2, Wfc, bfc):
    src = edge_index[0]
    dst = edge_index[1]
    src2 = jnp.pad(jnp.concatenate([src, src + N]), (0, 2 * CHUNK))
    ones_c = jnp.ones((CHUNK,), jnp.float32)
    zeros_c = jnp.zeros((CHUNK,), jnp.float32)

    degp = _sc_degree(dst, ones_c, zeros_c).reshape(2, N, 1)
    y1 = _tc_matmul_scale(x, W1, degp)
    aggy1 = _sc_agg(y1.reshape(2 * N, HALF), src2, dst).reshape(2, N, HALF)
    y2 = _tc_mid(aggy1, degp, b1.reshape(1, FH), W2)
    aggy2 = _sc_agg(y2.reshape(2 * N, HALF), src2, dst).reshape(2, N, HALF)

    return _tc_pool(aggy2, degp, b2.reshape(1, FH),
                    batch.reshape(N, 1), Wfc, bfc.reshape(1, C))


# speedup vs baseline: 1.0185x; 1.0076x over previous
import functools

import jax
import jax.numpy as jnp
from jax import lax
from jax.experimental import pallas as pl
from jax.experimental.pallas import tpu as pltpu
from jax.experimental.pallas import tpu_sc as plsc

N = 10000
E = 160000
FH = 256
HALF = 128
G = 64
C = 2

CHUNK = 128
NCHUNK = E // CHUNK
NSUB = 16
NCORE = 2
NRC = (N + CHUNK - 1) // CHUNK
TAIL = N - (NRC - 1) * CHUNK


def _sc_degree(dst, ones_c, zeros_c):
    mesh = plsc.VectorSubcoreMesh(core_axis_name="c", subcore_axis_name="s")
    n_per_core = NCHUNK // NCORE

    @functools.partial(
        pl.kernel,
        out_type=jax.ShapeDtypeStruct((NCORE * N,), jnp.float32),
        mesh=mesh,
        scratch_types=[
            pltpu.VMEM((CHUNK,), jnp.int32),
            pltpu.VMEM((CHUNK,), jnp.int32),
            pltpu.VMEM((CHUNK,), jnp.int32),
            pltpu.VMEM((CHUNK,), jnp.int32),
            pltpu.VMEM((CHUNK,), jnp.float32),
            pltpu.VMEM((CHUNK,), jnp.float32),
            pltpu.VMEM_SHARED((N,), jnp.float32),
            pltpu.SemaphoreType.DMA,
            pltpu.SemaphoreType.DMA,
            pltpu.SemaphoreType.DMA,
            pltpu.SemaphoreType.DMA,
            pltpu.SemaphoreType.DMA,
            pltpu.SemaphoreType.DMA,
        ],
    )
    def k(dst_hbm, ones_hbm, zeros_hbm, out_hbm, didx0, didx1, didx2, didx3,
          ones_v, stage_v, acc_sh, semd0, semd1, semd2, semd3, sems0, sems1):
        c = lax.axis_index("c")
        s = lax.axis_index("s")
        didx = (didx0, didx1, didx2, didx3)
        semd = (semd0, semd1, semd2, semd3)
        sems = (sems0, sems1)
        pltpu.sync_copy(ones_hbm, ones_v)
        pltpu.sync_copy(zeros_hbm, stage_v)

        def init_body(i, carry):
            rc = i * NSUB + s

            @pl.when(rc < NRC - 1)
            def _():
                pltpu.sync_copy(stage_v, acc_sh.at[pl.ds(rc * CHUNK, CHUNK)])

            @pl.when(rc == NRC - 1)
            def _():
                pltpu.sync_copy(stage_v.at[pl.ds(0, TAIL)],
                                acc_sh.at[pl.ds((NRC - 1) * CHUNK, TAIL)])

            return carry

        lax.fori_loop(0, (NRC + NSUB - 1) // NSUB, init_body, 0)
        plsc.subcore_barrier()

        n_iter = (n_per_core + NSUB - 1) // NSUB

        for b in range(2):
            @pl.when(b * NSUB + s < n_per_core)
            def _(b=b):
                e0 = (c * n_per_core + b * NSUB + s) * CHUNK
                pltpu.async_copy(dst_hbm.at[pl.ds(e0, CHUNK)], didx[b],
                                 semd[b])

        for j in range(n_iter):
            d = j % 4
            b = j % 2
            cid = j * NSUB + s

            @pl.when(cid < n_per_core)
            def _(d=d, b=b, j=j, cid=cid):
                pltpu.make_async_copy(dst_hbm.at[pl.ds(0, CHUNK)], didx[d],
                                      semd[d]).wait()
                if j >= 2:
                    pltpu.make_async_copy(ones_v, acc_sh.at[didx[d]],
                                          sems[b]).wait()
                pltpu.async_copy(ones_v, acc_sh.at[didx[d]], sems[b],
                                 add=True)
                nxt = (j + 2) * NSUB + s

                @pl.when(nxt < n_per_core)
                def _():
                    e0 = (c * n_per_core + nxt) * CHUNK
                    pltpu.async_copy(dst_hbm.at[pl.ds(e0, CHUNK)],
                                     didx[(j + 2) % 4], semd[(j + 2) % 4])

        for j in range(n_iter):
            b = j % 2
            cid = j * NSUB + s
            last = (j + 2) * NSUB + s >= n_per_core

            @pl.when((cid < n_per_core) & last)
            def _(b=b):
                pltpu.make_async_copy(ones_v, acc_sh.at[didx[0]],
                                      sems[b]).wait()

        plsc.subcore_barrier()

        def out_body(i, carry):
            rc = i * NSUB + s

            @pl.when(rc < NRC - 1)
            def _():
                pltpu.sync_copy(acc_sh.at[pl.ds(rc * CHUNK, CHUNK)], stage_v)
                pltpu.sync_copy(stage_v,
                                out_hbm.at[pl.ds(c * N + rc * CHUNK, CHUNK)])

            @pl.when(rc == NRC - 1)
            def _():
                r0 = (NRC - 1) * CHUNK
                pltpu.sync_copy(acc_sh.at[pl.ds(r0, TAIL)],
                                stage_v.at[pl.ds(0, TAIL)])
                pltpu.sync_copy(stage_v.at[pl.ds(0, TAIL)],
                                out_hbm.at[pl.ds(c * N + r0, TAIL)])

            return carry

        lax.fori_loop(0, (NRC + NSUB - 1) // NSUB, out_body, 0)

    return k(dst, ones_c, zeros_c)


PHASE = 40
CPT = NCHUNK // NSUB


def _sc_agg(y_flat, src2, dst):
    mesh = plsc.VectorSubcoreMesh(core_axis_name="c", subcore_axis_name="s")

    @functools.partial(
        pl.kernel,
        out_type=jax.ShapeDtypeStruct((NCORE * N, HALF), jnp.float32),
        mesh=mesh,
        scratch_types=[
            pltpu.VMEM((PHASE * CHUNK,), jnp.int32),
            pltpu.VMEM((CHUNK,), jnp.int32),
            pltpu.VMEM((CHUNK,), jnp.int32),
            pltpu.VMEM((CHUNK, HALF), jnp.float32),
            pltpu.VMEM((CHUNK, HALF), jnp.float32),
            pltpu.VMEM_SHARED((N, HALF), jnp.float32),
            pltpu.SemaphoreType.DMA,
            pltpu.SemaphoreType.DMA,
            pltpu.SemaphoreType.DMA,
            pltpu.SemaphoreType.DMA,
        ],
    )
    def k(y_hbm, src_hbm, dst_hbm, out_hbm,
          sidx_v, didx0, didx1, rows0, rows1, acc_sh,
          semg0, semg1, semd0, semd1):
        c = lax.axis_index("c")
        s = lax.axis_index("s")
        start = s * CPT + jnp.minimum(s, 2)
        count = jnp.where(s < 2, CPT + 1, CPT)
        rows = (rows0, rows1)
        didx = (didx0, didx1)
        semg = (semg0, semg1)
        semd = (semd0, semd1)

        n_iter_rc = (NRC + NSUB - 1) // NSUB

        for j in range(n_iter_rc):
            b = j % 2
            rc = j * NSUB + s

            @pl.when(rc < NRC - 1)
            def _(b=b, j=j, rc=rc):
                if j >= 2:
                    pltpu.make_async_copy(rows[b],
                                          acc_sh.at[pl.ds(0, CHUNK)],
                                          semd[b]).wait()
                pltpu.sync_copy(y_hbm.at[pl.ds(c * N + rc * CHUNK, CHUNK)],
                                rows[b])
                pltpu.async_copy(rows[b], acc_sh.at[pl.ds(rc * CHUNK,
                                                          CHUNK)],
                                 semd[b])

        for j in range(n_iter_rc):
            b = j % 2
            rc = j * NSUB + s
            done = (j + 2) * NSUB + s >= NRC - 1

            @pl.when((rc < NRC - 1) & done)
            def _(b=b):
                pltpu.make_async_copy(rows[b], acc_sh.at[pl.ds(0, CHUNK)],
                                      semd[b]).wait()

        @pl.when(s == (NRC - 1) % NSUB)
        def _():
            r0 = (NRC - 1) * CHUNK
            pltpu.sync_copy(y_hbm.at[pl.ds(c * N + r0, TAIL)],
                            rows0.at[pl.ds(0, TAIL)])
            pltpu.sync_copy(rows0.at[pl.ds(0, TAIL)],
                            acc_sh.at[pl.ds(r0, TAIL)])

        plsc.subcore_barrier()

        def do_phase(p):
            base = start + p * PHASE
            cnt = jnp.clip(count - p * PHASE, 0, PHASE)
            pltpu.sync_copy(
                src_hbm.at[pl.ds((c * NCHUNK + base) * CHUNK, PHASE * CHUNK)],
                sidx_v)

            for b in range(2):
                @pl.when(cnt > b)
                def _(b=b):
                    pltpu.async_copy(dst_hbm.at[pl.ds((base + b) * CHUNK,
                                                      CHUNK)],
                                     didx[b], semd[b])
                    pltpu.async_copy(
                        y_hbm.at[sidx_v.at[pl.ds(b * CHUNK, CHUNK)]],
                        rows[b], semg[b])

            def group(g, carry):
                for b in range(2):
                    i = g * 2 + b

                    @pl.when(cnt > i)
                    def _(b=b, i=i):
                        pltpu.make_async_copy(
                            dst_hbm.at[pl.ds(0, CHUNK)], didx[b],
                            semd[b]).wait()
                        pltpu.make_async_copy(
                            y_hbm.at[sidx_v.at[pl.ds(0, CHUNK)]],
                            rows[b], semg[b]).wait()
                        pltpu.sync_copy(rows[b], acc_sh.at[didx[b]],
                                        add=True)

                    @pl.when(cnt > i + 2)
                    def _(b=b, i=i):
                        pltpu.async_copy(
                            dst_hbm.at[pl.ds((base + i + 2) * CHUNK, CHUNK)],
                            didx[b], semd[b])
                        pltpu.async_copy(
                            y_hbm.at[sidx_v.at[pl.ds((i + 2) * CHUNK,
                                                     CHUNK)]],
                            rows[b], semg[b])

                return carry

            lax.fori_loop(0, PHASE // 2, group, 0)

        do_phase(0)
        do_phase(1)
        plsc.subcore_barrier()

        n_full = NRC - 1
        for j in range(n_iter_rc):
            b = j % 2
            rc = j * NSUB + s

            @pl.when(rc < n_full)
            def _(b=b, j=j, rc=rc):
                if j >= 2:
                    pltpu.make_async_copy(rows[b],
                                          out_hbm.at[pl.ds(0, CHUNK)],
                                          semg[b]).wait()
                pltpu.sync_copy(acc_sh.at[pl.ds(rc * CHUNK, CHUNK)], rows[b])
                pltpu.async_copy(rows[b],
                                 out_hbm.at[pl.ds(c * N + rc * CHUNK,
                                                  CHUNK)],
                                 semg[b])

        for j in range(n_iter_rc):
            b = j % 2
            rc = j * NSUB + s
            done = (j + 2) * NSUB + s >= n_full

            @pl.when((rc < n_full) & done)
            def _(b=b):
                pltpu.make_async_copy(rows[b], out_hbm.at[pl.ds(0, CHUNK)],
                                      semg[b]).wait()

        @pl.when(s == (NRC - 1) % NSUB)
        def _():
            r0 = (NRC - 1) * CHUNK
            pltpu.sync_copy(acc_sh.at[pl.ds(r0, TAIL)],
                            rows0.at[pl.ds(0, TAIL)])
            pltpu.sync_copy(rows0.at[pl.ds(0, TAIL)],
                            out_hbm.at[pl.ds(c * N + r0, TAIL)])

    return k(y_flat, src2, dst)


BR = 2000
NB = N // BR


def _dinv_block(d_ref):
    deg = d_ref[0] + d_ref[1] + 1.0
    return lax.rsqrt(jnp.maximum(deg, 1.0))


def _tc_matmul_scale(xin, W, degp):
    def body(x_ref, w_ref, d_ref, o_ref):
        dinv = _dinv_block(d_ref)
        xw = jnp.dot(x_ref[...], w_ref[...],
                     preferred_element_type=jnp.float32)
        o_ref[0] = xw[:, :HALF] * dinv
        o_ref[1] = xw[:, HALF:] * dinv

    return pl.pallas_call(
        body,
        grid=(NB,),
        in_specs=[
            pl.BlockSpec((BR, FH), lambda r: (r, 0)),
            pl.BlockSpec((FH, FH), lambda r: (0, 0)),
            pl.BlockSpec((2, BR, 1), lambda r: (0, r, 0)),
        ],
        out_specs=pl.BlockSpec((2, BR, HALF), lambda r: (0, r, 0)),
        out_shape=jax.ShapeDtypeStruct((2, N, HALF), jnp.float32),
    )(xin, W, degp)


def _tc_mid(aggy, degp, b, W):
    def body(a_ref, d_ref, b_ref, w_ref, o_ref):
        dinv = _dinv_block(d_ref)
        lo = dinv * a_ref[0] + b_ref[:, :HALF]
        hi = dinv * a_ref[1] + b_ref[:, HALF:]
        h = jnp.maximum(jnp.concatenate([lo, hi], axis=1), 0.0)
        xw = jnp.dot(h, w_ref[...], preferred_element_type=jnp.float32)
        o_ref[0] = xw[:, :HALF] * dinv
        o_ref[1] = xw[:, HALF:] * dinv

    return pl.pallas_call(
        body,
        grid=(NB,),
        in_specs=[
            pl.BlockSpec((2, BR, HALF), lambda r: (0, r, 0)),
            pl.BlockSpec((2, BR, 1), lambda r: (0, r, 0)),
            pl.BlockSpec((1, FH), lambda r: (0, 0)),
            pl.BlockSpec((FH, FH), lambda r: (0, 0)),
        ],
        out_specs=pl.BlockSpec((2, BR, HALF), lambda r: (0, r, 0)),
        out_shape=jax.ShapeDtypeStruct((2, N, HALF), jnp.float32),
    )(aggy, degp, b, W)


def _tc_pool(aggy, degp, b, batch2, Wfc, bfc):
    def body(a_ref, d_ref, b_ref, bt_ref, wfc_ref, bfc_ref,
             o_ref, acc, cnt):
        r = pl.program_id(0)

        @pl.when(r == 0)
        def _():
            acc[...] = jnp.zeros((G, FH), jnp.float32)
            cnt[...] = jnp.zeros((G, 1), jnp.float32)

        dinv = _dinv_block(d_ref)
        lo = jnp.maximum(dinv * a_ref[0] + b_ref[:, :HALF], 0.0)
        hi = jnp.maximum(dinv * a_ref[1] + b_ref[:, HALF:], 0.0)
        h = jnp.concatenate([lo, hi], axis=1)
        gids = lax.broadcasted_iota(jnp.int32, (BR, G), 1)
        oh = (bt_ref[...] == gids).astype(jnp.float32)
        acc[...] += lax.dot_general(oh, h, (((0,), (0,)), ((), ())),
                                    preferred_element_type=jnp.float32)
        cnt[...] += lax.dot_general(oh, jnp.ones((BR, 1), jnp.float32),
                                    (((0,), (0,)), ((), ())),
                                    preferred_element_type=jnp.float32)

        @pl.when(r == NB - 1)
        def _():
            pooled = acc[...] / jnp.maximum(cnt[...], 1.0)
            logits = jnp.dot(pooled, wfc_ref[...],
                             preferred_element_type=jnp.float32) + bfc_ref[...]
            m = jnp.max(logits, axis=1, keepdims=True)
            z = logits - m
            o_ref[...] = z - jnp.log(jnp.sum(jnp.exp(z), axis=1,
                                             keepdims=True))

    return pl.pallas_call(
        body,
        grid=(NB,),
        in_specs=[
            pl.BlockSpec((2, BR, HALF), lambda r: (0, r, 0)),
            pl.BlockSpec((2, BR, 1), lambda r: (0, r, 0)),
            pl.BlockSpec((1, FH), lambda r: (0, 0)),
            pl.BlockSpec((BR, 1), lambda r: (r, 0)),
            pl.BlockSpec((FH, C), lambda r: (0, 0)),
            pl.BlockSpec((1, C), lambda r: (0, 0)),
        ],
        out_specs=pl.BlockSpec((G, C), lambda r: (0, 0)),
        out_shape=jax.ShapeDtypeStruct((G, C), jnp.float32),
        scratch_shapes=[
            pltpu.VMEM((G, FH), jnp.float32),
            pltpu.VMEM((G, 1), jnp.float32),
        ],
    )(aggy, degp, b, batch2, Wfc, bfc)


def kernel(x, edge_index, batch, W1, b1, W2, b2, Wfc, bfc):
    src = edge_index[0]
    dst = edge_index[1]
    src2 = jnp.pad(jnp.concatenate([src, src + N]), (0, 2 * CHUNK))
    ones_c = jnp.ones((CHUNK,), jnp.float32)
    zeros_c = jnp.zeros((CHUNK,), jnp.float32)

    degp = _sc_degree(dst, ones_c, zeros_c).reshape(2, N, 1)
    y1 = _tc_matmul_scale(x, W1, degp)
    aggy1 = _sc_agg(y1.reshape(2 * N, HALF), src2, dst).reshape(2, N, HALF)
    y2 = _tc_mid(aggy1, degp, b1.reshape(1, FH), W2)
    aggy2 = _sc_agg(y2.reshape(2 * N, HALF), src2, dst).reshape(2, N, HALF)

    return _tc_pool(aggy2, degp, b2.reshape(1, FH),
                    batch.reshape(N, 1), Wfc, bfc.reshape(1, C))
